# baseline (speedup 1.0000x reference)
import functools

import jax
import jax.numpy as jnp
from jax import lax
from jax.experimental import pallas as pl
from jax.experimental.pallas import tpu as pltpu
from jax.experimental.pallas import tpu_sc as plsc

_L = 2048
_V = 257
_D = 64
_GROWS = 4360
_SMIN = -(_L + 127)
_SMAX = 128

_NC = 2
_NS = 16
_NW = _NC * _NS
_ROWS_PER_W = _L // _NW


def _build_g8(table):
    def body(tab_ref, g8_ref):
        t0 = tab_ref[0:1, :]
        t256 = tab_ref[256:257, :]
        for k in range(8):
            a = 2048 - k
            b = 2304 - k
            g8_ref[k, 0:a, :] = jnp.broadcast_to(t0, (a, _D))
            g8_ref[k, a:b, :] = tab_ref[1:257, :]
            g8_ref[k, b:_GROWS, :] = jnp.broadcast_to(t256, (_GROWS - b, _D))

    return pl.pallas_call(
        body,
        out_shape=jax.ShapeDtypeStruct((8, _GROWS, _D), jnp.float32),
    )(table)


@functools.partial(
    pl.kernel,
    mesh=plsc.VectorSubcoreMesh(core_axis_name="c", subcore_axis_name="s"),
    out_type=jax.ShapeDtypeStruct((_L, _L, _D), jnp.float32),
    scratch_types=[
        pltpu.VMEM((16,), jnp.int32),
        pltpu.SemaphoreType.DMA,
    ],
)
def _sc_expand(g8_hbm, delta_hbm, out_hbm, delta_v, sem):
    cid = lax.axis_index("c")
    sid = lax.axis_index("s")

    pltpu.sync_copy(delta_hbm, delta_v)
    delta = delta_v[...][0]

    wid = cid * _NS + sid
    base = wid * _ROWS_PER_W

    def fire(r, carry):
        i = base + r
        start = jnp.clip(delta - i, _SMIN, _SMAX) - _SMIN
        k = lax.rem(start, 8)
        q = pl.multiple_of(start - k, 8)
        pltpu.make_async_copy(
            g8_hbm.at[k, pl.ds(q, _L)], out_hbm.at[i], sem).start()
        return carry

    lax.fori_loop(0, _ROWS_PER_W, fire, 0)

    def drain(r, carry):
        pltpu.make_async_copy(
            g8_hbm.at[0, pl.ds(0, _L)], out_hbm.at[base], sem).wait()
        return carry

    lax.fori_loop(0, _ROWS_PER_W, drain, 0)


def kernel(length_q, length_k, embeddings_table):
    delta = (jnp.asarray(length_k, jnp.int32)
             - jnp.asarray(length_q, jnp.int32))
    delta16 = jnp.broadcast_to(delta.reshape(1), (16,))
    g8 = _build_g8(embeddings_table)
    return _sc_expand(g8, delta16)


# speedup vs baseline: 31.7269x; 31.7269x over previous
import functools

import jax
import jax.numpy as jnp
from jax import lax
from jax.experimental import pallas as pl
from jax.experimental.pallas import tpu as pltpu
from jax.experimental.pallas import tpu_sc as plsc

_L = 2048
_V = 257
_D = 64
_GROWS = 4424
_SMIN = -(_L + 127)
_SMAX = 128

_NC = 2
_NS = 16
_NW = _NC * _NS
_ROWS_PER_W = _L // _NW
_CHUNK = 512
_NCHUNK = _L // _CHUNK
_SPAN = 584


def _build_g(table):
    def body(tab_ref, g_ref):
        t0 = tab_ref[0:1, :]
        t256 = tab_ref[256:257, :]
        g_ref[0:2048, :] = jnp.broadcast_to(t0, (2048, _D))
        g_ref[2048:2304, :] = tab_ref[1:257, :]
        g_ref[2304:_GROWS, :] = jnp.broadcast_to(t256, (_GROWS - 2304, _D))

    return pl.pallas_call(
        body,
        out_shape=jax.ShapeDtypeStruct((_GROWS, _D), jnp.float32),
    )(table)


@functools.partial(
    pl.kernel,
    mesh=plsc.VectorSubcoreMesh(core_axis_name="c", subcore_axis_name="s"),
    out_type=jax.ShapeDtypeStruct((_L, _L, _D), jnp.float32),
    scratch_types=[
        pltpu.VMEM((_SPAN, _D), jnp.float32),
        pltpu.VMEM((16,), jnp.int32),
        pltpu.SemaphoreType.DMA,
    ],
)
def _sc_expand(g_hbm, delta_hbm, out_hbm, buf, delta_v, sem):
    cid = lax.axis_index("c")
    sid = lax.axis_index("s")

    pltpu.sync_copy(delta_hbm, delta_v)
    delta = delta_v[...][0]

    wid = cid * _NS + sid
    base = wid * _ROWS_PER_W

    def start_of(i):
        return jnp.clip(delta - i, _SMIN, _SMAX) - _SMIN

    smin = start_of(base + _ROWS_PER_W - 1)
    lo0 = pl.multiple_of(smin & ~jnp.int32(7), 8)

    def chunk_pass(c, carry):
        lo = pl.multiple_of(lo0 + c * _CHUNK, 8)
        pltpu.sync_copy(g_hbm.at[pl.ds(lo, _SPAN)], buf)

        def fire(r, cr):
            i = base + r
            off = start_of(i) + c * _CHUNK - lo
            pltpu.make_async_copy(
                buf.at[pl.ds(off, _CHUNK)],
                out_hbm.at[i, pl.ds(c * _CHUNK, _CHUNK)],
                sem).start()
            return cr

        lax.fori_loop(0, _ROWS_PER_W, fire, 0)

        def drain(r, cr):
            pltpu.make_async_copy(
                buf.at[pl.ds(0, _CHUNK)],
                out_hbm.at[base, pl.ds(0, _CHUNK)],
                sem).wait()
            return cr

        lax.fori_loop(0, _ROWS_PER_W, drain, 0)
        return carry

    lax.fori_loop(0, _NCHUNK, chunk_pass, 0)


def kernel(length_q, length_k, embeddings_table):
    delta = (jnp.asarray(length_k, jnp.int32)
             - jnp.asarray(length_q, jnp.int32))
    delta16 = jnp.broadcast_to(delta.reshape(1), (16,))
    g = _build_g(embeddings_table)
    return _sc_expand(g, delta16)
